# baseline (speedup 1.0000x reference)
import functools

import jax
import jax.numpy as jnp
from jax import lax
from jax.experimental import pallas as pl
from jax.experimental.pallas import tpu as pltpu
from jax.experimental.pallas import tpu_sc as plsc

V = 1000000
D = 64
B = 16384
NNEG = 20

NC = 2
NS = 16
L = 16
NW = NC * NS
NB = B // NW
C = 32
NCHUNK = NB // C
NEGC = C * NNEG
ROWS = NNEG + 1
ROWS_PAD = 24
W2 = 2 * D

CB = 8192
PB = CB // 2
GP = -(-V // CB)
NP_PAD = GP * PB


def _tc_pack_body(x_ref, o_ref):
    xt = x_ref[...].T
    x3 = xt.reshape(PB, 2, D)
    o_ref[...] = jnp.concatenate([x3[:, 0, :], x3[:, 1, :]], axis=1)


_tc_pack = pl.pallas_call(
    _tc_pack_body,
    grid=(GP,),
    in_specs=[pl.BlockSpec((D, CB), lambda j: (0, j))],
    out_specs=pl.BlockSpec((PB, W2), lambda j: (j, 0)),
    out_shape=jax.ShapeDtypeStruct((NP_PAD, W2), jnp.float32),
)


def _sc_body(center_hbm, pos_hbm, neg_hbm, in_tab, out_tab, scores_hbm,
             cen_idx, pos_idx, neg_idx, cen_pair, pos_pair, neg_pair,
             cen_rows, pos_rows, neg_rows, scores_v, sem):
    wid = lax.axis_index("s") * NC + lax.axis_index("c")
    base = pl.multiple_of(wid * NB, NB)

    pltpu.sync_copy(center_hbm.at[pl.ds(base, NB)], cen_idx)
    pltpu.sync_copy(pos_hbm.at[pl.ds(base, NB)], pos_idx)
    pltpu.sync_copy(neg_hbm.at[pl.ds(base * NNEG, NB * NNEG)], neg_idx)

    lane = lax.iota(jnp.int32, L)

    def pair_cp(i, carry):
        off = pl.multiple_of(i * L, L)
        cen_pair[pl.ds(off, L)] = cen_idx[pl.ds(off, L)] >> 1
        pos_pair[pl.ds(off, L)] = pos_idx[pl.ds(off, L)] >> 1
        return carry

    lax.fori_loop(0, NB // L, pair_cp, 0)

    def pair_np(i, carry):
        off = pl.multiple_of(i * L, L)
        neg_pair[pl.ds(off, L)] = neg_idx[pl.ds(off, L)] >> 1
        return carry

    lax.fori_loop(0, NB * NNEG // L, pair_np, 0)

    def chunk_body(g, carry):
        goff = pl.multiple_of(g * C, C)
        copies = [
            pltpu.async_copy(in_tab.at[cen_pair.at[pl.ds(goff, C)]],
                             cen_rows, sem),
            pltpu.async_copy(out_tab.at[pos_pair.at[pl.ds(goff, C)]],
                             pos_rows, sem),
        ]
        for k in range(NEGC // 128):
            copies.append(pltpu.async_copy(
                out_tab.at[neg_pair.at[pl.ds(goff * NNEG + k * 128, 128)]],
                neg_rows.at[pl.ds(k * 128, 128)], sem))
        for cp in copies:
            cp.wait()

        for t in range(C // L):
            r = t * L + lane
            rn = [r * NNEG + n for n in range(NNEG)]
            ccol = (cen_idx[pl.ds(goff + t * L, L)] & 1) * D
            pcol = (pos_idx[pl.ds(goff + t * L, L)] & 1) * D
            ncol = [(plsc.load_gather(neg_idx,
                                      [(goff + t * L + lane) * NNEG + n])
                     & 1) * D for n in range(NNEG)]

            def d_body(dd, accs):
                dv = jnp.broadcast_to(dd, (L,)).astype(jnp.int32)
                cen_d = plsc.load_gather(cen_rows, [r, ccol + dv])
                pos_d = plsc.load_gather(pos_rows, [r, pcol + dv])
                new = [accs[0] + cen_d * pos_d]
                for n in range(NNEG):
                    neg_d = plsc.load_gather(neg_rows, [rn[n], ncol[n] + dv])
                    new.append(accs[n + 1] + cen_d * neg_d)
                return tuple(new)

            accs = lax.fori_loop(
                0, D, d_body,
                tuple(jnp.zeros((L,), jnp.float32) for _ in range(ROWS)))
            col = goff + t * L
            scores_v[0, pl.ds(col, L)] = accs[0]
            for n in range(NNEG):
                scores_v[1 + n, pl.ds(col, L)] = -accs[1 + n]
        return carry

    lax.fori_loop(0, NCHUNK, chunk_body, 0)

    pad = jnp.full((L,), 40.0, jnp.float32)
    for j in range(ROWS, ROWS_PAD):
        for c0 in range(0, NB, L):
            scores_v[j, pl.ds(c0, L)] = pad

    pltpu.sync_copy(scores_v, scores_hbm.at[:, pl.ds(base, NB)])


_sc_scores = functools.partial(
    pl.kernel,
    out_type=jax.ShapeDtypeStruct((ROWS_PAD, B), jnp.float32),
    mesh=plsc.VectorSubcoreMesh(core_axis_name="c", subcore_axis_name="s"),
    scratch_types=[
        pltpu.VMEM((NB,), jnp.int32),
        pltpu.VMEM((NB,), jnp.int32),
        pltpu.VMEM((NB * NNEG,), jnp.int32),
        pltpu.VMEM((NB,), jnp.int32),
        pltpu.VMEM((NB,), jnp.int32),
        pltpu.VMEM((NB * NNEG,), jnp.int32),
        pltpu.VMEM((C, W2), jnp.float32),
        pltpu.VMEM((C, W2), jnp.float32),
        pltpu.VMEM((NEGC, W2), jnp.float32),
        pltpu.VMEM((ROWS_PAD, NB), jnp.float32),
        pltpu.SemaphoreType.DMA,
    ],
    compiler_params=pltpu.CompilerParams(needs_layout_passes=False,
                                         use_tc_tiling_on_sc=True),
)(_sc_body)


def _tc_loss_body(scores_ref, out_ref):
    x = scores_ref[...]
    row = lax.broadcasted_iota(jnp.int32, x.shape, 0)
    val = -jnp.log(jax.nn.sigmoid(x) + 1e-10)
    out_ref[0, 0] = jnp.sum(jnp.where(row < ROWS, val, 0.0)) / B


_tc_loss = pl.pallas_call(
    _tc_loss_body,
    out_shape=jax.ShapeDtypeStruct((1, 1), jnp.float32),
    in_specs=[pl.BlockSpec(memory_space=pltpu.VMEM)],
    out_specs=pl.BlockSpec(memory_space=pltpu.SMEM),
)


def kernel(center, pos, neg, input_table, output_table):
    in_pairs = _tc_pack(input_table.T)
    out_pairs = _tc_pack(output_table.T)
    scores = _sc_scores(center.astype(jnp.int32), pos.astype(jnp.int32),
                        neg.reshape(-1).astype(jnp.int32),
                        in_pairs, out_pairs)
    return _tc_loss(scores)[0, 0]


# speedup vs baseline: 1.0694x; 1.0694x over previous
import functools

import jax
import jax.numpy as jnp
from jax import lax
from jax.experimental import pallas as pl
from jax.experimental.pallas import tpu as pltpu
from jax.experimental.pallas import tpu_sc as plsc

V = 1000000
D = 64
B = 16384
NNEG = 20

NC = 2
NS = 16
L = 16
NW = NC * NS
NB = B // NW
C = 32
NCHUNK = NB // C
NEGC = C * NNEG
ROWS = NNEG + 1
ROWS_PAD = 24
W2 = 2 * D

CB = 8192
PB = CB // 2
GP = -(-V // CB)
NP_PAD = GP * PB


def _tc_pack_body(x_ref, o_ref):
    xt = x_ref[...].T
    x3 = xt.reshape(PB, 2, D)
    o_ref[...] = jnp.concatenate([x3[:, 0, :], x3[:, 1, :]], axis=1)


_tc_pack = pl.pallas_call(
    _tc_pack_body,
    grid=(GP,),
    in_specs=[pl.BlockSpec((D, CB), lambda j: (0, j))],
    out_specs=pl.BlockSpec((PB, W2), lambda j: (j, 0)),
    out_shape=jax.ShapeDtypeStruct((NP_PAD, W2), jnp.float32),
)


def _sc_body(center_hbm, pos_hbm, neg_hbm, in_tab, out_tab, scores_hbm,
             cen_idx, pos_idx, neg_idx, cen_pair, pos_pair, neg_pair,
             cen_rows, pos_rows, neg_rows, cen_t, scores_v, sem):
    wid = lax.axis_index("s") * NC + lax.axis_index("c")
    base = pl.multiple_of(wid * NB, NB)

    pltpu.sync_copy(center_hbm.at[pl.ds(base, NB)], cen_idx)
    pltpu.sync_copy(pos_hbm.at[pl.ds(base, NB)], pos_idx)
    pltpu.sync_copy(neg_hbm.at[pl.ds(base * NNEG, NB * NNEG)], neg_idx)

    lane = lax.iota(jnp.int32, L)

    def pair_cp(i, carry):
        off = pl.multiple_of(i * L, L)
        cen_pair[pl.ds(off, L)] = cen_idx[pl.ds(off, L)] >> 1
        pos_pair[pl.ds(off, L)] = pos_idx[pl.ds(off, L)] >> 1
        return carry

    lax.fori_loop(0, NB // L, pair_cp, 0)

    def pair_np(i, carry):
        off = pl.multiple_of(i * L, L)
        neg_pair[pl.ds(off, L)] = neg_idx[pl.ds(off, L)] >> 1
        return carry

    lax.fori_loop(0, NB * NNEG // L, pair_np, 0)

    def chunk_body(g, carry):
        goff = pl.multiple_of(g * C, C)
        copies = [
            pltpu.async_copy(in_tab.at[cen_pair.at[pl.ds(goff, C)]],
                             cen_rows, sem),
            pltpu.async_copy(out_tab.at[pos_pair.at[pl.ds(goff, C)]],
                             pos_rows, sem),
        ]
        for k in range(NEGC // 128):
            copies.append(pltpu.async_copy(
                out_tab.at[neg_pair.at[pl.ds(goff * NNEG + k * 128, 128)]],
                neg_rows.at[pl.ds(k * 128, 128)], sem))
        for cp in copies:
            cp.wait()

        for t in range(C // L):
            r = t * L + lane
            col = goff + t * L
            ccol = (cen_idx[pl.ds(col, L)] & 1) * D
            pcol = (pos_idx[pl.ds(col, L)] & 1) * D

            def p1_body(i, acc):
                for j in range(4):
                    dd = i * 4 + j
                    dv = jnp.broadcast_to(dd, (L,)).astype(jnp.int32)
                    cen_d = plsc.load_gather(cen_rows, [r, ccol + dv])
                    pos_d = plsc.load_gather(pos_rows, [r, pcol + dv])
                    cen_t[pl.ds(dd * L, L)] = cen_d
                    acc = acc + cen_d * pos_d
                return acc

            scores_v[0, pl.ds(col, L)] = lax.fori_loop(
                0, D // 4, p1_body, jnp.zeros((L,), jnp.float32))

            for n in range(NNEG):
                rn = r * NNEG + n
                ncol = (plsc.load_gather(neg_idx,
                                         [(goff + r) * NNEG + n]) & 1) * D

                def p2_body(i, acc, rn=rn, ncol=ncol):
                    for j in range(8):
                        dd = i * 8 + j
                        dv = jnp.broadcast_to(dd, (L,)).astype(jnp.int32)
                        neg_d = plsc.load_gather(neg_rows, [rn, ncol + dv])
                        acc = acc + neg_d * cen_t[pl.ds(dd * L, L)]
                    return acc

                scores_v[1 + n, pl.ds(col, L)] = -lax.fori_loop(
                    0, D // 8, p2_body, jnp.zeros((L,), jnp.float32))
        return carry

    lax.fori_loop(0, NCHUNK, chunk_body, 0)

    pad = jnp.full((L,), 40.0, jnp.float32)
    for j in range(ROWS, ROWS_PAD):
        for c0 in range(0, NB, L):
            scores_v[j, pl.ds(c0, L)] = pad

    pltpu.sync_copy(scores_v, scores_hbm.at[:, pl.ds(base, NB)])


_sc_scores = functools.partial(
    pl.kernel,
    out_type=jax.ShapeDtypeStruct((ROWS_PAD, B), jnp.float32),
    mesh=plsc.VectorSubcoreMesh(core_axis_name="c", subcore_axis_name="s"),
    scratch_types=[
        pltpu.VMEM((NB,), jnp.int32),
        pltpu.VMEM((NB,), jnp.int32),
        pltpu.VMEM((NB * NNEG,), jnp.int32),
        pltpu.VMEM((NB,), jnp.int32),
        pltpu.VMEM((NB,), jnp.int32),
        pltpu.VMEM((NB * NNEG,), jnp.int32),
        pltpu.VMEM((C, W2), jnp.float32),
        pltpu.VMEM((C, W2), jnp.float32),
        pltpu.VMEM((NEGC, W2), jnp.float32),
        pltpu.VMEM((D * L,), jnp.float32),
        pltpu.VMEM((ROWS_PAD, NB), jnp.float32),
        pltpu.SemaphoreType.DMA,
    ],
    compiler_params=pltpu.CompilerParams(needs_layout_passes=False,
                                         use_tc_tiling_on_sc=True),
)(_sc_body)


def _tc_loss_body(scores_ref, out_ref):
    x = scores_ref[...]
    row = lax.broadcasted_iota(jnp.int32, x.shape, 0)
    val = -jnp.log(jax.nn.sigmoid(x) + 1e-10)
    out_ref[0, 0] = jnp.sum(jnp.where(row < ROWS, val, 0.0)) / B


_tc_loss = pl.pallas_call(
    _tc_loss_body,
    out_shape=jax.ShapeDtypeStruct((1, 1), jnp.float32),
    in_specs=[pl.BlockSpec(memory_space=pltpu.VMEM)],
    out_specs=pl.BlockSpec(memory_space=pltpu.SMEM),
)


def kernel(center, pos, neg, input_table, output_table):
    in_pairs = _tc_pack(input_table.T)
    out_pairs = _tc_pack(output_table.T)
    scores = _sc_scores(center.astype(jnp.int32), pos.astype(jnp.int32),
                        neg.reshape(-1).astype(jnp.int32),
                        in_pairs, out_pairs)
    return _tc_loss(scores)[0, 0]


# speedup vs baseline: 1.4084x; 1.3170x over previous
import functools

import jax
import jax.numpy as jnp
from jax import lax
from jax.experimental import pallas as pl
from jax.experimental.pallas import tpu as pltpu
from jax.experimental.pallas import tpu_sc as plsc

V = 1000000
D = 64
B = 16384
NNEG = 20

NC = 2
NS = 16
L = 16
NW = NC * NS
NB = B // NW
C = 32
NCHUNK = NB // C
NEGC = C * NNEG
ROWS = NNEG + 1
ROWS_PAD = 24
W2 = 2 * D

CB = 8192
PB = CB // 2
GP = -(-V // CB)
NP_PAD = GP * PB


def _tc_pack_body(x_ref, o_ref):
    xt = x_ref[...].T
    x3 = xt.reshape(PB, 2, D)
    o_ref[...] = jnp.concatenate([x3[:, 0, :], x3[:, 1, :]], axis=1)


_tc_pack = pl.pallas_call(
    _tc_pack_body,
    grid=(GP,),
    in_specs=[pl.BlockSpec((D, CB), lambda j: (0, j))],
    out_specs=pl.BlockSpec((PB, W2), lambda j: (j, 0)),
    out_shape=jax.ShapeDtypeStruct((NP_PAD, W2), jnp.float32),
)


def _sc_body(center_hbm, pos_hbm, neg_hbm, in_tab, out_tab, scores_hbm,
             cen_idx, pos_idx, neg_idx, cen_pair, pos_pair, neg_pair,
             cen_rows, pos_rows, neg_rows, cen_t, scores_v, sem):
    wid = lax.axis_index("s") * NC + lax.axis_index("c")
    base = pl.multiple_of(wid * NB, NB)

    pltpu.sync_copy(center_hbm.at[pl.ds(base, NB)], cen_idx.at[pl.ds(0, NB)])
    pltpu.sync_copy(pos_hbm.at[pl.ds(base, NB)], pos_idx.at[pl.ds(0, NB)])
    pltpu.sync_copy(neg_hbm.at[pl.ds(base * NNEG, NB * NNEG)],
                    neg_idx.at[pl.ds(0, NB * NNEG)])

    lane = lax.iota(jnp.int32, L)

    def pair_cp(i, carry):
        off = pl.multiple_of(i * L, L)
        cen_pair[pl.ds(off, L)] = cen_idx[pl.ds(off, L)] >> 1
        pos_pair[pl.ds(off, L)] = pos_idx[pl.ds(off, L)] >> 1
        return carry

    lax.fori_loop(0, NB // L, pair_cp, 0)

    def pair_np(i, carry):
        off = pl.multiple_of(i * L, L)
        neg_pair[pl.ds(off, L)] = neg_idx[pl.ds(off, L)] >> 1
        return carry

    lax.fori_loop(0, NB * NNEG // L, pair_np, 0)

    def chunk_body(g, carry):
        goff = pl.multiple_of(g * C, C)
        copies = [
            pltpu.async_copy(in_tab.at[cen_pair.at[pl.ds(goff, C)]],
                             cen_rows, sem),
            pltpu.async_copy(out_tab.at[pos_pair.at[pl.ds(goff, C)]],
                             pos_rows, sem),
        ]
        for k in range(NEGC // 128):
            copies.append(pltpu.async_copy(
                out_tab.at[neg_pair.at[pl.ds(goff * NNEG + k * 128, 128)]],
                neg_rows.at[pl.ds(k * 128, 128)], sem))
        for cp in copies:
            cp.wait()

        for t in range(C // L):
            col = goff + t * L

            def b_body(jb, scs):
                row = t * L + jb
                msk = lane == jnp.broadcast_to(jb, (L,)).astype(jnp.int32)
                cbase = (cen_idx[pl.ds(col + jb, L)][0] & 1) * D
                pbase = (pos_idx[pl.ds(col + jb, L)][0] & 1) * D
                nb20 = (goff + row) * NNEG
                na = neg_idx[pl.ds(nb20, L)]
                nb = neg_idx[pl.ds(nb20 + 4, L)]
                c = [cen_rows[row, pl.ds(cbase + L * j, L)] for j in range(4)]
                p = [pos_rows[row, pl.ds(pbase + L * j, L)] for j in range(4)]
                s = jnp.sum(c[0] * p[0] + c[1] * p[1] + c[2] * p[2]
                            + c[3] * p[3])
                out = [jnp.where(msk, s, scs[0])]
                for n in range(NNEG):
                    nbase = ((na[n] if n < L else nb[n - 4]) & 1) * D
                    nrow = row * NNEG + n
                    q = [neg_rows[nrow, pl.ds(nbase + L * j, L)]
                         for j in range(4)]
                    sn = jnp.sum(c[0] * q[0] + c[1] * q[1] + c[2] * q[2]
                                 + c[3] * q[3])
                    out.append(jnp.where(msk, -sn, scs[n + 1]))
                return tuple(out)

            scs = lax.fori_loop(
                0, L, b_body,
                tuple(jnp.zeros((L,), jnp.float32) for _ in range(ROWS)))
            for j in range(ROWS):
                scores_v[j, pl.ds(col, L)] = scs[j]
        return carry

    lax.fori_loop(0, NCHUNK, chunk_body, 0)

    pad = jnp.full((L,), 40.0, jnp.float32)
    for j in range(ROWS, ROWS_PAD):
        for c0 in range(0, NB, L):
            scores_v[j, pl.ds(c0, L)] = pad

    pltpu.sync_copy(scores_v, scores_hbm.at[:, pl.ds(base, NB)])


_sc_scores = functools.partial(
    pl.kernel,
    out_type=jax.ShapeDtypeStruct((ROWS_PAD, B), jnp.float32),
    mesh=plsc.VectorSubcoreMesh(core_axis_name="c", subcore_axis_name="s"),
    scratch_types=[
        pltpu.VMEM((NB + L,), jnp.int32),
        pltpu.VMEM((NB + L,), jnp.int32),
        pltpu.VMEM((NB * NNEG + L,), jnp.int32),
        pltpu.VMEM((NB,), jnp.int32),
        pltpu.VMEM((NB,), jnp.int32),
        pltpu.VMEM((NB * NNEG,), jnp.int32),
        pltpu.VMEM((C, W2), jnp.float32),
        pltpu.VMEM((C, W2), jnp.float32),
        pltpu.VMEM((NEGC, W2), jnp.float32),
        pltpu.VMEM((D * L,), jnp.float32),
        pltpu.VMEM((ROWS_PAD, NB), jnp.float32),
        pltpu.SemaphoreType.DMA,
    ],
    compiler_params=pltpu.CompilerParams(needs_layout_passes=False,
                                         use_tc_tiling_on_sc=True),
)(_sc_body)


def _tc_loss_body(scores_ref, out_ref):
    x = scores_ref[...]
    row = lax.broadcasted_iota(jnp.int32, x.shape, 0)
    val = -jnp.log(jax.nn.sigmoid(x) + 1e-10)
    out_ref[0, 0] = jnp.sum(jnp.where(row < ROWS, val, 0.0)) / B


_tc_loss = pl.pallas_call(
    _tc_loss_body,
    out_shape=jax.ShapeDtypeStruct((1, 1), jnp.float32),
    in_specs=[pl.BlockSpec(memory_space=pltpu.VMEM)],
    out_specs=pl.BlockSpec(memory_space=pltpu.SMEM),
)


def kernel(center, pos, neg, input_table, output_table):
    in_pairs = _tc_pack(input_table.T)
    out_pairs = _tc_pack(output_table.T)
    scores = _sc_scores(center.astype(jnp.int32), pos.astype(jnp.int32),
                        neg.reshape(-1).astype(jnp.int32),
                        in_pairs, out_pairs)
    return _tc_loss(scores)[0, 0]


# speedup vs baseline: 1.5686x; 1.1137x over previous
import functools

import jax
import jax.numpy as jnp
from jax import lax
from jax.experimental import pallas as pl
from jax.experimental.pallas import tpu as pltpu
from jax.experimental.pallas import tpu_sc as plsc

V = 1000000
D = 64
B = 16384
NNEG = 20

NC = 2
NS = 16
L = 16
NW = NC * NS
NB = B // NW
C = 16
NCHUNK = NB // C
NEGC = C * NNEG
ROWS = NNEG + 1
ROWS_PAD = 24
W2 = 2 * D

CB = 16384
PB = CB // 2
GP = -(-V // CB)
NP_PAD = GP * PB


def _tc_pack_body(x_ref, o_ref):
    x = x_ref[...]
    sel = jnp.concatenate([jnp.eye(D, dtype=jnp.float32)] * 2, axis=1)
    xt = lax.dot_general(x, sel, (((0,), (0,)), ((), ())),
                         preferred_element_type=jnp.float32)
    x3 = xt.reshape(PB, 2, W2)
    o_ref[...] = jnp.where(
        lax.broadcasted_iota(jnp.int32, (PB, W2), 1) < D,
        x3[:, 0, :], x3[:, 1, :])


_tc_pack = pl.pallas_call(
    _tc_pack_body,
    grid=(GP,),
    in_specs=[pl.BlockSpec((D, CB), lambda j: (0, j))],
    out_specs=pl.BlockSpec((PB, W2), lambda j: (j, 0)),
    out_shape=jax.ShapeDtypeStruct((NP_PAD, W2), jnp.float32),
)


def _sc_body(center_hbm, pos_hbm, neg_hbm, in_tab, out_tab, scores_hbm,
             cen_idx, pos_idx, neg_idx, cen_pair, pos_pair, neg_pair,
             cen_rows0, pos_rows0, neg_rows0, cen_rows1, pos_rows1,
             neg_rows1, scores_v, sem0, sem1):
    wid = lax.axis_index("s") * NC + lax.axis_index("c")
    base = pl.multiple_of(wid * NB, NB)

    pltpu.sync_copy(center_hbm.at[pl.ds(base, NB)], cen_idx.at[pl.ds(0, NB)])
    pltpu.sync_copy(pos_hbm.at[pl.ds(base, NB)], pos_idx.at[pl.ds(0, NB)])
    pltpu.sync_copy(neg_hbm.at[pl.ds(base * NNEG, NB * NNEG)],
                    neg_idx.at[pl.ds(0, NB * NNEG)])

    lane = lax.iota(jnp.int32, L)

    def pair_cp(i, carry):
        off = pl.multiple_of(i * L, L)
        cen_pair[pl.ds(off, L)] = cen_idx[pl.ds(off, L)] >> 1
        pos_pair[pl.ds(off, L)] = pos_idx[pl.ds(off, L)] >> 1
        return carry

    lax.fori_loop(0, NB // L, pair_cp, 0)

    def pair_np(i, carry):
        off = pl.multiple_of(i * L, L)
        neg_pair[pl.ds(off, L)] = neg_idx[pl.ds(off, L)] >> 1
        return carry

    lax.fori_loop(0, NB * NNEG // L, pair_np, 0)

    neg_cuts = [(0, 128), (128, 128), (256, 64)]

    def chunk_dma(g, cen_b, pos_b, neg_b, sm):
        goff = pl.multiple_of(g * C, C)
        yield pltpu.make_async_copy(
            in_tab.at[cen_pair.at[pl.ds(goff, C)]], cen_b, sm)
        yield pltpu.make_async_copy(
            out_tab.at[pos_pair.at[pl.ds(goff, C)]], pos_b, sm)
        for off, ln in neg_cuts:
            yield pltpu.make_async_copy(
                out_tab.at[neg_pair.at[pl.ds(goff * NNEG + off, ln)]],
                neg_b.at[pl.ds(off, ln)], sm)

    def start_chunk(g, bufs):
        for h in chunk_dma(g, *bufs):
            h.start()

    def wait_chunk(g, bufs):
        for h in chunk_dma(g, *bufs):
            h.wait()

    def compute_chunk(g, cen_b, pos_b, neg_b, sm, carry):
        goff = pl.multiple_of(g * C, C)
        cen_rows, pos_rows, neg_rows = cen_b, pos_b, neg_b
        for t in range(C // L):
            col = goff + t * L

            def b_body(jb, scs):
                row = t * L + jb
                msk = lane == jnp.broadcast_to(jb, (L,)).astype(jnp.int32)
                cbase = (cen_idx[pl.ds(col + jb, L)][0] & 1) * D
                pbase = (pos_idx[pl.ds(col + jb, L)][0] & 1) * D
                nb20 = (goff + row) * NNEG
                na = neg_idx[pl.ds(nb20, L)]
                nb = neg_idx[pl.ds(nb20 + 4, L)]
                c = [cen_rows[row, pl.ds(cbase + L * j, L)] for j in range(4)]
                p = [pos_rows[row, pl.ds(pbase + L * j, L)] for j in range(4)]
                s = jnp.sum(c[0] * p[0] + c[1] * p[1] + c[2] * p[2]
                            + c[3] * p[3])
                out = [jnp.where(msk, s, scs[0])]
                for n in range(NNEG):
                    nbase = ((na[n] if n < L else nb[n - 4]) & 1) * D
                    nrow = row * NNEG + n
                    q = [neg_rows[nrow, pl.ds(nbase + L * j, L)]
                         for j in range(4)]
                    sn = jnp.sum(c[0] * q[0] + c[1] * q[1] + c[2] * q[2]
                                 + c[3] * q[3])
                    out.append(jnp.where(msk, -sn, scs[n + 1]))
                return tuple(out)

            scs = lax.fori_loop(
                0, L, b_body,
                tuple(jnp.zeros((L,), jnp.float32) for _ in range(ROWS)))
            for j in range(ROWS):
                scores_v[j, pl.ds(col, L)] = scs[j]
        return carry

    set0 = (cen_rows0, pos_rows0, neg_rows0, sem0)
    set1 = (cen_rows1, pos_rows1, neg_rows1, sem1)
    start_chunk(0, set0)

    def pair_body(g2, carry):
        g0 = g2 * 2
        start_chunk(g0 + 1, set1)
        wait_chunk(g0, set0)
        compute_chunk(g0, *set0, carry)
        start_chunk(jnp.minimum(g0 + 2, NCHUNK - 1), set0)
        wait_chunk(g0 + 1, set1)
        compute_chunk(g0 + 1, *set1, carry)
        return carry

    lax.fori_loop(0, NCHUNK // 2, pair_body, 0)
    wait_chunk(NCHUNK - 1, set0)

    pad = jnp.full((L,), 40.0, jnp.float32)
    for j in range(ROWS, ROWS_PAD):
        for c0 in range(0, NB, L):
            scores_v[j, pl.ds(c0, L)] = pad

    pltpu.sync_copy(scores_v, scores_hbm.at[:, pl.ds(base, NB)])


_sc_scores = functools.partial(
    pl.kernel,
    out_type=jax.ShapeDtypeStruct((ROWS_PAD, B), jnp.float32),
    mesh=plsc.VectorSubcoreMesh(core_axis_name="c", subcore_axis_name="s"),
    scratch_types=[
        pltpu.VMEM((NB + L,), jnp.int32),
        pltpu.VMEM((NB + L,), jnp.int32),
        pltpu.VMEM((NB * NNEG + L,), jnp.int32),
        pltpu.VMEM((NB,), jnp.int32),
        pltpu.VMEM((NB,), jnp.int32),
        pltpu.VMEM((NB * NNEG,), jnp.int32),
        pltpu.VMEM((C, W2), jnp.float32),
        pltpu.VMEM((C, W2), jnp.float32),
        pltpu.VMEM((NEGC, W2), jnp.float32),
        pltpu.VMEM((C, W2), jnp.float32),
        pltpu.VMEM((C, W2), jnp.float32),
        pltpu.VMEM((NEGC, W2), jnp.float32),
        pltpu.VMEM((ROWS_PAD, NB), jnp.float32),
        pltpu.SemaphoreType.DMA,
        pltpu.SemaphoreType.DMA,
    ],
    compiler_params=pltpu.CompilerParams(needs_layout_passes=False,
                                         use_tc_tiling_on_sc=True),
)(_sc_body)


def _tc_loss_body(scores_ref, out_ref):
    x = scores_ref[...]
    row = lax.broadcasted_iota(jnp.int32, x.shape, 0)
    val = -jnp.log(jax.nn.sigmoid(x) + 1e-10)
    out_ref[0, 0] = jnp.sum(jnp.where(row < ROWS, val, 0.0)) / B


_tc_loss = pl.pallas_call(
    _tc_loss_body,
    out_shape=jax.ShapeDtypeStruct((1, 1), jnp.float32),
    in_specs=[pl.BlockSpec(memory_space=pltpu.VMEM)],
    out_specs=pl.BlockSpec(memory_space=pltpu.SMEM),
)


def kernel(center, pos, neg, input_table, output_table):
    in_pairs = _tc_pack(input_table.T)
    out_pairs = _tc_pack(output_table.T)
    scores = _sc_scores(center.astype(jnp.int32), pos.astype(jnp.int32),
                        neg.reshape(-1).astype(jnp.int32),
                        in_pairs, out_pairs)
    return _tc_loss(scores)[0, 0]


# speedup vs baseline: 1.5889x; 1.0130x over previous
import functools

import jax
import jax.numpy as jnp
from jax import lax
from jax.experimental import pallas as pl
from jax.experimental.pallas import tpu as pltpu
from jax.experimental.pallas import tpu_sc as plsc

V = 1000000
D = 64
B = 16384
NNEG = 20

NC = 2
NS = 16
L = 16
NW = NC * NS
NB = B // NW
C = 16
NCHUNK = NB // C
NEGC = C * NNEG
ROWS = NNEG + 1
ROWS_PAD = 24
W2 = 2 * D

CB = 32768
PB = CB // 2
GP = -(-V // CB)
NP_PAD = GP * PB


def _tc_pack_body(x_ref, o_ref):
    x = x_ref[...]
    sel = jnp.concatenate([jnp.eye(D, dtype=jnp.float32)] * 2, axis=1)
    xt = lax.dot_general(x, sel, (((0,), (0,)), ((), ())),
                         preferred_element_type=jnp.float32)
    x3 = xt.reshape(PB, 2, W2)
    o_ref[...] = jnp.where(
        lax.broadcasted_iota(jnp.int32, (PB, W2), 1) < D,
        x3[:, 0, :], x3[:, 1, :])


_tc_pack = pl.pallas_call(
    _tc_pack_body,
    grid=(GP,),
    in_specs=[pl.BlockSpec((D, CB), lambda j: (0, j))],
    out_specs=pl.BlockSpec((PB, W2), lambda j: (j, 0)),
    out_shape=jax.ShapeDtypeStruct((NP_PAD, W2), jnp.float32),
)


def _sc_body(center_hbm, pos_hbm, neg_hbm, in_tab, out_tab, scores_hbm,
             cen_idx, pos_idx, neg_idx, cen_pair, pos_pair, neg_pair,
             cen_rows0, pos_rows0, neg_rows0, cen_rows1, pos_rows1,
             neg_rows1, scores_v, sem0, sem1):
    wid = lax.axis_index("s") * NC + lax.axis_index("c")
    base = pl.multiple_of(wid * NB, NB)

    pltpu.sync_copy(center_hbm.at[pl.ds(base, NB)], cen_idx.at[pl.ds(0, NB)])
    pltpu.sync_copy(pos_hbm.at[pl.ds(base, NB)], pos_idx.at[pl.ds(0, NB)])
    pltpu.sync_copy(neg_hbm.at[pl.ds(base * NNEG, NB * NNEG)],
                    neg_idx.at[pl.ds(0, NB * NNEG)])

    lane = lax.iota(jnp.int32, L)

    def pair_cp(i, carry):
        off = pl.multiple_of(i * L, L)
        cen_pair[pl.ds(off, L)] = cen_idx[pl.ds(off, L)] >> 1
        pos_pair[pl.ds(off, L)] = pos_idx[pl.ds(off, L)] >> 1
        return carry

    lax.fori_loop(0, NB // L, pair_cp, 0)

    def pair_np(i, carry):
        off = pl.multiple_of(i * L, L)
        neg_pair[pl.ds(off, L)] = neg_idx[pl.ds(off, L)] >> 1
        return carry

    lax.fori_loop(0, NB * NNEG // L, pair_np, 0)

    neg_cuts = [(0, 128), (128, 128), (256, 64)]

    def chunk_dma(g, cen_b, pos_b, neg_b, sm):
        goff = pl.multiple_of(g * C, C)
        yield pltpu.make_async_copy(
            in_tab.at[cen_pair.at[pl.ds(goff, C)]], cen_b, sm)
        yield pltpu.make_async_copy(
            out_tab.at[pos_pair.at[pl.ds(goff, C)]], pos_b, sm)
        for off, ln in neg_cuts:
            yield pltpu.make_async_copy(
                out_tab.at[neg_pair.at[pl.ds(goff * NNEG + off, ln)]],
                neg_b.at[pl.ds(off, ln)], sm)

    def start_chunk(g, bufs):
        for h in chunk_dma(g, *bufs):
            h.start()

    def wait_chunk(g, bufs):
        for h in chunk_dma(g, *bufs):
            h.wait()

    def compute_chunk(g, cen_b, pos_b, neg_b, sm, carry):
        goff = pl.multiple_of(g * C, C)
        cen_rows, pos_rows, neg_rows = cen_b, pos_b, neg_b
        for t in range(C // L):
            col = goff + t * L

            def b_body(jb, scs):
                row = t * L + jb
                msk = lane == jnp.broadcast_to(jb, (L,)).astype(jnp.int32)
                cbase = (cen_idx[pl.ds(col + jb, L)][0] & 1) * D
                pbase = (pos_idx[pl.ds(col + jb, L)][0] & 1) * D
                nb20 = (goff + row) * NNEG
                na = neg_idx[pl.ds(nb20, L)]
                nb = neg_idx[pl.ds(nb20 + 4, L)]
                c = [cen_rows[row, pl.ds(cbase + L * j, L)] for j in range(4)]
                p = [pos_rows[row, pl.ds(pbase + L * j, L)] for j in range(4)]
                s = jnp.sum(c[0] * p[0] + c[1] * p[1] + c[2] * p[2]
                            + c[3] * p[3])
                out = [jnp.where(msk, s, scs[0])]
                for n in range(NNEG):
                    nbase = ((na[n] if n < L else nb[n - 4]) & 1) * D
                    nrow = row * NNEG + n
                    q = [neg_rows[nrow, pl.ds(nbase + L * j, L)]
                         for j in range(4)]
                    sn = jnp.sum(c[0] * q[0] + c[1] * q[1] + c[2] * q[2]
                                 + c[3] * q[3])
                    out.append(jnp.where(msk, -sn, scs[n + 1]))
                return tuple(out)

            scs = lax.fori_loop(
                0, L, b_body,
                tuple(jnp.zeros((L,), jnp.float32) for _ in range(ROWS)))
            for j in range(ROWS):
                scores_v[j, pl.ds(col, L)] = scs[j]
        return carry

    set0 = (cen_rows0, pos_rows0, neg_rows0, sem0)
    set1 = (cen_rows1, pos_rows1, neg_rows1, sem1)
    start_chunk(0, set0)

    def pair_body(g2, carry):
        g0 = g2 * 2
        start_chunk(g0 + 1, set1)
        wait_chunk(g0, set0)
        compute_chunk(g0, *set0, carry)
        start_chunk(jnp.minimum(g0 + 2, NCHUNK - 1), set0)
        wait_chunk(g0 + 1, set1)
        compute_chunk(g0 + 1, *set1, carry)
        return carry

    lax.fori_loop(0, NCHUNK // 2, pair_body, 0)
    wait_chunk(NCHUNK - 1, set0)

    pad = jnp.full((L,), 40.0, jnp.float32)
    for j in range(ROWS, ROWS_PAD):
        for c0 in range(0, NB, L):
            scores_v[j, pl.ds(c0, L)] = pad

    pltpu.sync_copy(scores_v, scores_hbm.at[:, pl.ds(base, NB)])


_sc_scores = functools.partial(
    pl.kernel,
    out_type=jax.ShapeDtypeStruct((ROWS_PAD, B), jnp.float32),
    mesh=plsc.VectorSubcoreMesh(core_axis_name="c", subcore_axis_name="s"),
    scratch_types=[
        pltpu.VMEM((NB + L,), jnp.int32),
        pltpu.VMEM((NB + L,), jnp.int32),
        pltpu.VMEM((NB * NNEG + L,), jnp.int32),
        pltpu.VMEM((NB,), jnp.int32),
        pltpu.VMEM((NB,), jnp.int32),
        pltpu.VMEM((NB * NNEG,), jnp.int32),
        pltpu.VMEM((C, W2), jnp.float32),
        pltpu.VMEM((C, W2), jnp.float32),
        pltpu.VMEM((NEGC, W2), jnp.float32),
        pltpu.VMEM((C, W2), jnp.float32),
        pltpu.VMEM((C, W2), jnp.float32),
        pltpu.VMEM((NEGC, W2), jnp.float32),
        pltpu.VMEM((ROWS_PAD, NB), jnp.float32),
        pltpu.SemaphoreType.DMA,
        pltpu.SemaphoreType.DMA,
    ],
    compiler_params=pltpu.CompilerParams(needs_layout_passes=False,
                                         use_tc_tiling_on_sc=True),
)(_sc_body)


def _tc_loss_body(scores_ref, out_ref):
    x = scores_ref[...]
    row = lax.broadcasted_iota(jnp.int32, x.shape, 0)
    val = -jnp.log(jax.nn.sigmoid(x) + 1e-10)
    out_ref[0, 0] = jnp.sum(jnp.where(row < ROWS, val, 0.0)) / B


_tc_loss = pl.pallas_call(
    _tc_loss_body,
    out_shape=jax.ShapeDtypeStruct((1, 1), jnp.float32),
    in_specs=[pl.BlockSpec(memory_space=pltpu.VMEM)],
    out_specs=pl.BlockSpec(memory_space=pltpu.SMEM),
)


def kernel(center, pos, neg, input_table, output_table):
    in_pairs = _tc_pack(input_table.T)
    out_pairs = _tc_pack(output_table.T)
    scores = _sc_scores(center.astype(jnp.int32), pos.astype(jnp.int32),
                        neg.reshape(-1).astype(jnp.int32),
                        in_pairs, out_pairs)
    return _tc_loss(scores)[0, 0]
